# baseline (speedup 1.0000x reference)
import jax
import jax.numpy as jnp
from jax.experimental import pallas as pl
from jax.experimental.pallas import tpu as pltpu


def _ema_body(v_ref, o_ref):
    q_low = v_ref[0]
    q_high = v_ref[1]
    low = v_ref[2]
    high = v_ref[3]
    decay = jnp.float32(0.99)
    new_low = decay * low + (1.0 - decay) * q_low
    new_high = decay * high + (1.0 - decay) * q_high
    o_ref[0] = new_low
    o_ref[1] = jnp.maximum(jnp.float32(1.0), new_high - new_low)


def kernel(x, low, high):
    x = jax.lax.stop_gradient(x)
    q_low = jnp.quantile(x, 0.05)
    q_high = jnp.quantile(x, 0.95)
    vals = jnp.stack([q_low, q_high, low, high])
    out = pl.pallas_call(
        _ema_body,
        in_specs=[pl.BlockSpec(memory_space=pltpu.SMEM)],
        out_specs=pl.BlockSpec(memory_space=pltpu.SMEM),
        out_shape=jax.ShapeDtypeStruct((2,), jnp.float32),
    )(vals)
    return (out[0], out[1])


# speedup vs baseline: 15.1718x; 15.1718x over previous
import functools

import jax
import jax.numpy as jnp
import numpy as np
from jax import lax
from jax.experimental import pallas as pl
from jax.experimental.pallas import tpu as pltpu
from jax.experimental.pallas import tpu_sc as plsc

N = 128 * 32768
NC, NS = 2, 16
NW = NC * NS
PER_W = N // NW
CHUNK = 16384
NCHUNK = PER_W // CHUNK
VECS = CHUNK // 16

B1 = 4096
SHIFT1 = 20
H1 = B1 * 16
NT = 4
B2 = 1024
SHIFT2 = 10
H2 = NT * B2 * 16

_SIGN = np.uint32(0x80000000)


def _mono_key(v):
    u = plsc.bitcast(v, jnp.uint32)
    return jnp.where(u >= _SIGN, ~u, u | _SIGN)


def _zero_hist(hist_ref, nwords):
    z = jnp.zeros((16,), jnp.int32)

    def body(i, c):
        hist_ref[pl.ds(i * 16, 16)] = z
        return c

    lax.fori_loop(0, nwords // 16, body, 0)


def _pass1_body(x_hbm, out_hbm, hist, buf):
    c = lax.axis_index("c")
    s = lax.axis_index("s")
    wid = s * NC + c
    lane = lax.iota(jnp.int32, 16)
    ones = jnp.ones((16,), jnp.int32)
    _zero_hist(hist, H1)
    base = wid * PER_W

    def chunk_body(ci, carry):
        pltpu.sync_copy(x_hbm.at[pl.ds(base + ci * CHUNK, CHUNK)], buf)

        def vec_body(i, c2):
            k = _mono_key(buf[pl.ds(i * 16, 16)])
            bucket = (k >> jnp.uint32(SHIFT1)).astype(jnp.int32)
            idx = (bucket << 4) + lane
            plsc.addupdate_scatter(hist, [idx], ones)
            return c2

        lax.fori_loop(0, VECS, vec_body, 0)
        return carry

    lax.fori_loop(0, NCHUNK, chunk_body, 0)
    pltpu.sync_copy(hist, out_hbm.at[pl.ds(wid * H1, H1)])


def _pass2_body(x_hbm, pref_hbm, out_hbm, hist, buf, pbuf):
    c = lax.axis_index("c")
    s = lax.axis_index("s")
    wid = s * NC + c
    lane = lax.iota(jnp.int32, 16)
    ones = jnp.ones((16,), jnp.int32)
    _zero_hist(hist, H2)
    pltpu.sync_copy(pref_hbm, pbuf)
    prefs = [pbuf[t] for t in range(NT)]
    base = wid * PER_W

    def chunk_body(ci, carry):
        pltpu.sync_copy(x_hbm.at[pl.ds(base + ci * CHUNK, CHUNK)], buf)

        def vec_body(i, c2):
            k = _mono_key(buf[pl.ds(i * 16, 16)])
            hi = (k >> jnp.uint32(SHIFT1)).astype(jnp.int32)
            bucket = ((k >> jnp.uint32(SHIFT2)) & jnp.uint32(B2 - 1)).astype(
                jnp.int32)
            b16 = (bucket << 4) + lane
            for t in range(NT):
                plsc.addupdate_scatter(hist, [b16 + t * B2 * 16], ones,
                                       mask=hi == prefs[t])
            return c2

        lax.fori_loop(0, VECS, vec_body, 0)
        return carry

    lax.fori_loop(0, NCHUNK, chunk_body, 0)
    pltpu.sync_copy(hist, out_hbm.at[pl.ds(wid * H2, H2)])


_mesh = plsc.VectorSubcoreMesh(core_axis_name="c", subcore_axis_name="s")

_pass1 = functools.partial(
    pl.kernel,
    out_type=jax.ShapeDtypeStruct((NW * H1,), jnp.int32),
    scratch_types=[
        pltpu.VMEM((H1,), jnp.int32),
        pltpu.VMEM((CHUNK,), jnp.float32),
    ],
    mesh=_mesh,
    compiler_params=pltpu.CompilerParams(needs_layout_passes=False),
)(_pass1_body)

_pass2 = functools.partial(
    pl.kernel,
    out_type=jax.ShapeDtypeStruct((NW * H2,), jnp.int32),
    scratch_types=[
        pltpu.VMEM((H2,), jnp.int32),
        pltpu.VMEM((CHUNK,), jnp.float32),
        pltpu.VMEM((NT, 16), jnp.int32),
    ],
    mesh=_mesh,
    compiler_params=pltpu.CompilerParams(needs_layout_passes=False),
)(_pass2_body)

_POS_LO = 0.05 * (N - 1)
_POS_HI = 0.95 * (N - 1)
_K_LO = int(_POS_LO)
_K_HI = int(_POS_HI)
_F_LO = _POS_LO - _K_LO
_F_HI = _POS_HI - _K_HI
_RANKS = (_K_LO, _K_LO + 1, _K_HI, _K_HI + 1)


def kernel(x, low, high):
    x = lax.stop_gradient(x)
    xf = x.reshape(-1)

    out1 = _pass1(xf)
    hist1 = out1.reshape(NW, B1, 16).sum(axis=(0, 2))
    c1 = jnp.cumsum(hist1)
    ranks = jnp.array(_RANKS, jnp.int32)
    p1 = jnp.sum(c1[None, :] <= ranks[:, None], axis=1).astype(jnp.int32)
    below = jnp.where(p1 > 0, c1[jnp.maximum(p1 - 1, 0)], 0)
    r = ranks - below

    prefs = jnp.broadcast_to(p1[:, None], (NT, 16)).astype(jnp.int32)
    out2 = _pass2(xf, prefs)
    hist2 = out2.reshape(NW, NT, B2, 16).sum(axis=(0, 3))
    c2 = jnp.cumsum(hist2, axis=1)
    b2 = jnp.sum(c2 <= r[:, None], axis=1).astype(jnp.uint32)

    key_mid = (((p1.astype(jnp.uint32) << 10) | b2) << 10) | jnp.uint32(512)
    orig = jnp.where(key_mid >= _SIGN, key_mid ^ _SIGN, ~key_mid)
    vals = lax.bitcast_convert_type(orig, jnp.float32)

    q_lo = vals[0] + jnp.float32(_F_LO) * (vals[1] - vals[0])
    q_hi = vals[2] + jnp.float32(_F_HI) * (vals[3] - vals[2])

    decay = jnp.float32(0.99)
    new_low = decay * low + (1.0 - decay) * q_lo
    new_high = decay * high + (1.0 - decay) * q_hi
    inverse_scale = jnp.maximum(jnp.float32(1.0), new_high - new_low)
    return (new_low, inverse_scale)


# speedup vs baseline: 23.2885x; 1.5350x over previous
import functools

import jax
import jax.numpy as jnp
import numpy as np
from jax import lax
from jax.experimental import pallas as pl
from jax.experimental.pallas import tpu as pltpu
from jax.experimental.pallas import tpu_sc as plsc

N = 128 * 32768
NC, NS = 2, 16
NW = NC * NS
PER_W = N // NW
CHUNK = 16384
NCHUNK = PER_W // CHUNK
VECS = CHUNK // 16
UNROLL = 4

B = 1024
SHIFT1 = 22
SHIFT2 = 12
NT = 4
H1 = 16 * B
H2 = NT * 16 * B

_SIGN = np.uint32(0x80000000)
_MININT = np.int32(-0x80000000)


def _mono_key(v):
    ki = plsc.bitcast(v, jnp.int32)
    flip = (ki >> 31) | _MININT
    return plsc.bitcast(ki ^ flip, jnp.uint32)


def _zero(ref, nwords):
    z = jnp.zeros((16,), jnp.int32)

    def body(i, c):
        for u in range(UNROLL):
            ref[pl.ds((i * UNROLL + u) * 16, 16)] = z
        return c

    lax.fori_loop(0, nwords // (16 * UNROLL), body, 0)


def _fold_lanes(hist, base):
    def body(j, c):
        off = base + j * 16
        acc = hist[pl.ds(off, 16)]
        for r in range(1, 16):
            acc = acc + hist[pl.ds(off + r * B, 16)]
        hist[pl.ds(off, 16)] = acc
        return c

    lax.fori_loop(0, B // 16, body, 0)


def _stream_chunks(x_hbm, buf, sems, base, compute_vec):
    def copy_in(ci, b):
        return pltpu.async_copy(
            x_hbm.at[pl.ds(base + ci * CHUNK, CHUNK)], buf.at[b], sems[b])

    handles = [copy_in(0, 0), copy_in(1, 1)]
    for ci in range(NCHUNK):
        b = ci % 2
        handles[b].wait()

        def vec_body(i, c, _b=b):
            for u in range(UNROLL):
                compute_vec(buf[_b, pl.ds((i * UNROLL + u) * 16, 16)])
            return c

        lax.fori_loop(0, VECS // UNROLL, vec_body, 0)
        if ci + 2 < NCHUNK:
            handles[b] = copy_in(ci + 2, b)


def _pass1_body(x_hbm, out_hbm, hist, buf, sem0, sem1):
    wid = lax.axis_index("s") * NC + lax.axis_index("c")
    lane_base = lax.iota(jnp.int32, 16) * B
    ones = jnp.ones((16,), jnp.int32)
    _zero(hist, H1)

    def compute_vec(v):
        key = _mono_key(v)
        bucket = plsc.bitcast(key >> np.uint32(SHIFT1), jnp.int32)
        plsc.addupdate_scatter(hist, [lane_base + bucket], ones)

    _stream_chunks(x_hbm, buf, (sem0, sem1), wid * PER_W, compute_vec)
    _fold_lanes(hist, 0)
    pltpu.sync_copy(hist.at[pl.ds(0, B)], out_hbm.at[pl.ds(wid * B, B)])


def _pass2_body(x_hbm, pref_hbm, out_hbm, hist, buf, pbuf, sem0, sem1):
    wid = lax.axis_index("s") * NC + lax.axis_index("c")
    lane_base = lax.iota(jnp.int32, 16) * B
    ones = jnp.ones((16,), jnp.int32)
    _zero(hist, H2)
    pltpu.sync_copy(pref_hbm, pbuf)
    prefs = [pbuf[t] for t in range(NT)]

    def compute_vec(v):
        key = _mono_key(v)
        hi = plsc.bitcast(key >> np.uint32(SHIFT1), jnp.int32)
        bucket = plsc.bitcast(
            (key >> np.uint32(SHIFT2)) & np.uint32(B - 1), jnp.int32)
        idx = lane_base + bucket
        for t in range(NT):
            plsc.addupdate_scatter(hist, [idx + t * 16 * B], ones,
                                   mask=hi == prefs[t])

    _stream_chunks(x_hbm, buf, (sem0, sem1), wid * PER_W, compute_vec)
    for t in range(NT):
        _fold_lanes(hist, t * 16 * B)
        pltpu.sync_copy(hist.at[pl.ds(t * 16 * B, B)],
                        out_hbm.at[pl.ds((wid * NT + t) * B, B)])


_mesh = plsc.VectorSubcoreMesh(core_axis_name="c", subcore_axis_name="s")
_params = pltpu.CompilerParams(needs_layout_passes=False)

_pass1 = functools.partial(
    pl.kernel,
    out_type=jax.ShapeDtypeStruct((NW * B,), jnp.int32),
    scratch_types=[
        pltpu.VMEM((H1,), jnp.int32),
        pltpu.VMEM((2, CHUNK), jnp.float32),
        pltpu.SemaphoreType.DMA,
        pltpu.SemaphoreType.DMA,
    ],
    mesh=_mesh,
    compiler_params=_params,
)(_pass1_body)

_pass2 = functools.partial(
    pl.kernel,
    out_type=jax.ShapeDtypeStruct((NW * NT * B,), jnp.int32),
    scratch_types=[
        pltpu.VMEM((H2,), jnp.int32),
        pltpu.VMEM((2, CHUNK), jnp.float32),
        pltpu.VMEM((NT, 16), jnp.int32),
        pltpu.SemaphoreType.DMA,
        pltpu.SemaphoreType.DMA,
    ],
    mesh=_mesh,
    compiler_params=_params,
)(_pass2_body)

_POS_LO = 0.05 * (N - 1)
_POS_HI = 0.95 * (N - 1)
_K_LO = int(_POS_LO)
_K_HI = int(_POS_HI)
_F_LO = _POS_LO - _K_LO
_F_HI = _POS_HI - _K_HI
_RANKS = (_K_LO, _K_LO + 1, _K_HI, _K_HI + 1)


def kernel(x, low, high):
    x = lax.stop_gradient(x)
    xf = x.reshape(-1)

    out1 = _pass1(xf)
    hist1 = out1.reshape(NW, B).sum(axis=0)
    c1 = jnp.cumsum(hist1)
    ranks = jnp.array(_RANKS, jnp.int32)
    p1 = jnp.sum(c1[None, :] <= ranks[:, None], axis=1).astype(jnp.int32)
    below = jnp.where(p1 > 0, c1[jnp.maximum(p1 - 1, 0)], 0)
    r = ranks - below

    prefs = jnp.broadcast_to(p1[:, None], (NT, 16)).astype(jnp.int32)
    out2 = _pass2(xf, prefs)
    hist2 = out2.reshape(NW, NT, B).sum(axis=0)
    c2 = jnp.cumsum(hist2, axis=1)
    b2 = jnp.sum(c2 <= r[:, None], axis=1).astype(jnp.uint32)

    key_mid = (((p1.astype(jnp.uint32) << 10) | b2) << 12) | jnp.uint32(2048)
    orig = jnp.where(key_mid >= _SIGN, key_mid ^ _SIGN, ~key_mid)
    vals = lax.bitcast_convert_type(orig, jnp.float32)

    q_lo = vals[0] + jnp.float32(_F_LO) * (vals[1] - vals[0])
    q_hi = vals[2] + jnp.float32(_F_HI) * (vals[3] - vals[2])

    decay = jnp.float32(0.99)
    new_low = decay * low + (1.0 - decay) * q_lo
    new_high = decay * high + (1.0 - decay) * q_hi
    inverse_scale = jnp.maximum(jnp.float32(1.0), new_high - new_low)
    return (new_low, inverse_scale)


# speedup vs baseline: 38.0820x; 1.6352x over previous
import functools

import jax
import jax.numpy as jnp
import numpy as np
from jax import lax
from jax.experimental import pallas as pl
from jax.experimental.pallas import tpu as pltpu
from jax.experimental.pallas import tpu_sc as plsc

N = 128 * 32768
NC, NS = 2, 16
NW = NC * NS
PER_W = N // NW
CHUNK = 16384
NCHUNK = PER_W // CHUNK
VECS = CHUNK // 16

B1 = 1024
SHIFT1 = 22
U1 = 4
H1 = U1 * 16 * B1

B2 = 512
SHIFT2 = 13
U2 = 2
NT = 4
H2 = NT * U2 * 16 * B2

_SIGN = np.uint32(0x80000000)
_MININT = np.int32(-0x80000000)


def _mono_key(v):
    ki = plsc.bitcast(v, jnp.int32)
    flip = (ki >> 31) | _MININT
    return plsc.bitcast(ki ^ flip, jnp.uint32)


def _zero(ref, nwords):
    z = jnp.zeros((16,), jnp.int32)

    @plsc.parallel_loop(0, nwords // 16)
    def _(i):
        ref[pl.ds(i * 16, 16)] = z


def _fold_rows(hist, base, nrows, width):
    @plsc.parallel_loop(0, width // 16)
    def _(j):
        off = base + j * 16
        acc = hist[pl.ds(off, 16)]
        for r in range(1, nrows):
            acc = acc + hist[pl.ds(off + r * width, 16)]
        hist[pl.ds(off, 16)] = acc


def _stream_chunks(x_hbm, buf, sems, base, unroll, compute_vec):
    def copy_in(ci, b):
        return pltpu.async_copy(
            x_hbm.at[pl.ds(base + ci * CHUNK, CHUNK)], buf.at[b], sems[b])

    handles = [copy_in(0, 0), copy_in(1, 1)]
    for ci in range(NCHUNK):
        b = ci % 2
        handles[b].wait()

        @plsc.parallel_loop(0, VECS // unroll, unroll=unroll)
        def _(i, _b=b):
            for u in range(unroll):
                compute_vec(u, buf[_b, pl.ds((i * unroll + u) * 16, 16)])

        if ci + 2 < NCHUNK:
            handles[b] = copy_in(ci + 2, b)


def _pass1_body(x_hbm, out_hbm, hist, buf, sem0, sem1):
    wid = lax.axis_index("s") * NC + lax.axis_index("c")
    lane_base = lax.iota(jnp.int32, 16) * B1
    ones = jnp.ones((16,), jnp.int32)
    _zero(hist, H1)

    def compute_vec(u, v):
        key = _mono_key(v)
        bucket = plsc.bitcast(key >> np.uint32(SHIFT1), jnp.int32)
        plsc.addupdate_scatter(hist, [(lane_base + u * 16 * B1) + bucket],
                               ones)

    _stream_chunks(x_hbm, buf, (sem0, sem1), wid * PER_W, U1, compute_vec)
    _fold_rows(hist, 0, U1 * 16, B1)
    pltpu.sync_copy(hist.at[pl.ds(0, B1)], out_hbm.at[pl.ds(wid * B1, B1)])


def _pass2_body(x_hbm, pref_hbm, out_hbm, hist, buf, pbuf, sem0, sem1):
    wid = lax.axis_index("s") * NC + lax.axis_index("c")
    lane_base = lax.iota(jnp.int32, 16) * B2
    ones = jnp.ones((16,), jnp.int32)
    _zero(hist, H2)
    pltpu.sync_copy(pref_hbm, pbuf)
    prefs = [pbuf[t] for t in range(NT)]

    def compute_vec(u, v):
        key = _mono_key(v)
        hi = plsc.bitcast(key >> np.uint32(SHIFT1), jnp.int32)
        bucket = plsc.bitcast(
            (key >> np.uint32(SHIFT2)) & np.uint32(B2 - 1), jnp.int32)
        idx = lane_base + bucket
        for t in range(NT):
            plsc.addupdate_scatter(
                hist, [idx + (t * U2 + u) * 16 * B2], ones,
                mask=hi == prefs[t])

    _stream_chunks(x_hbm, buf, (sem0, sem1), wid * PER_W, U2, compute_vec)
    for t in range(NT):
        _fold_rows(hist, t * U2 * 16 * B2, U2 * 16, B2)
        pltpu.sync_copy(hist.at[pl.ds(t * U2 * 16 * B2, B2)],
                        out_hbm.at[pl.ds((wid * NT + t) * B2, B2)])


_mesh = plsc.VectorSubcoreMesh(core_axis_name="c", subcore_axis_name="s")
_params = pltpu.CompilerParams(needs_layout_passes=False)

_pass1 = functools.partial(
    pl.kernel,
    out_type=jax.ShapeDtypeStruct((NW * B1,), jnp.int32),
    scratch_types=[
        pltpu.VMEM((H1,), jnp.int32),
        pltpu.VMEM((2, CHUNK), jnp.float32),
        pltpu.SemaphoreType.DMA,
        pltpu.SemaphoreType.DMA,
    ],
    mesh=_mesh,
    compiler_params=_params,
)(_pass1_body)

_pass2 = functools.partial(
    pl.kernel,
    out_type=jax.ShapeDtypeStruct((NW * NT * B2,), jnp.int32),
    scratch_types=[
        pltpu.VMEM((H2,), jnp.int32),
        pltpu.VMEM((2, CHUNK), jnp.float32),
        pltpu.VMEM((NT, 16), jnp.int32),
        pltpu.SemaphoreType.DMA,
        pltpu.SemaphoreType.DMA,
    ],
    mesh=_mesh,
    compiler_params=_params,
)(_pass2_body)

_POS_LO = 0.05 * (N - 1)
_POS_HI = 0.95 * (N - 1)
_K_LO = int(_POS_LO)
_K_HI = int(_POS_HI)
_F_LO = _POS_LO - _K_LO
_F_HI = _POS_HI - _K_HI
_RANKS = (_K_LO, _K_LO + 1, _K_HI, _K_HI + 1)


def kernel(x, low, high):
    x = lax.stop_gradient(x)
    xf = x.reshape(-1)

    out1 = _pass1(xf)
    hist1 = out1.reshape(NW, B1).sum(axis=0)
    c1 = jnp.cumsum(hist1)
    ranks = jnp.array(_RANKS, jnp.int32)
    p1 = jnp.sum(c1[None, :] <= ranks[:, None], axis=1).astype(jnp.int32)
    below = jnp.where(p1 > 0, c1[jnp.maximum(p1 - 1, 0)], 0)
    r = ranks - below

    prefs = jnp.broadcast_to(p1[:, None], (NT, 16)).astype(jnp.int32)
    out2 = _pass2(xf, prefs)
    hist2 = out2.reshape(NW, NT, B2).sum(axis=0)
    c2 = jnp.cumsum(hist2, axis=1)
    b2 = jnp.sum(c2 <= r[:, None], axis=1).astype(jnp.uint32)

    key_mid = ((((p1.astype(jnp.uint32) << 9) | b2) << 13)
               | jnp.uint32(1 << 12))
    orig = jnp.where(key_mid >= _SIGN, key_mid ^ _SIGN, ~key_mid)
    vals = lax.bitcast_convert_type(orig, jnp.float32)

    q_lo = vals[0] + jnp.float32(_F_LO) * (vals[1] - vals[0])
    q_hi = vals[2] + jnp.float32(_F_HI) * (vals[3] - vals[2])

    decay = jnp.float32(0.99)
    new_low = decay * low + (1.0 - decay) * q_lo
    new_high = decay * high + (1.0 - decay) * q_hi
    inverse_scale = jnp.maximum(jnp.float32(1.0), new_high - new_low)
    return (new_low, inverse_scale)


# speedup vs baseline: 49.0973x; 1.2893x over previous
import functools

import jax
import jax.numpy as jnp
import numpy as np
from jax import lax
from jax.experimental import pallas as pl
from jax.experimental.pallas import tpu as pltpu
from jax.experimental.pallas import tpu_sc as plsc

ROWS, COLS = 128, 32768
N = ROWS * COLS
NC, NS = 2, 16
NW = NC * NS
PER_W = N // NW
CHUNK = 16384
NCHUNK = PER_W // CHUNK
ROWCH = COLS // CHUNK
VECS = CHUNK // 16

B1 = 1024
SHIFT1 = 22
U1 = 4
H1 = U1 * 16 * B1

B2 = 512
SHIFT2 = 13
U2 = 2
NT = 4
REG = 16 * B2
H2 = (NT + 1) * U2 * REG

_SIGN = np.uint32(0x80000000)
_MININT = np.int32(-0x80000000)


def _mono_key(v):
    ki = plsc.bitcast(v, jnp.int32)
    flip = (ki >> 31) | _MININT
    return plsc.bitcast(ki ^ flip, jnp.uint32)


def _zero(ref, nwords):
    z = jnp.zeros((16,), jnp.int32)

    @plsc.parallel_loop(0, nwords // 16)
    def _(i):
        ref[pl.ds(i * 16, 16)] = z


def _fold_rows(hist, base, nrows, width):
    @plsc.parallel_loop(0, width // 16)
    def _(j):
        off = base + j * 16
        acc = hist[pl.ds(off, 16)]
        for r in range(1, nrows):
            acc = acc + hist[pl.ds(off + r * width, 16)]
        hist[pl.ds(off, 16)] = acc


def _stream_chunks(x_hbm, buf, sems, wid, unroll, compute_vec):
    def copy_in(ci, b):
        row = wid * (NCHUNK // ROWCH) + ci // ROWCH
        col = (ci % ROWCH) * CHUNK
        return pltpu.async_copy(
            x_hbm.at[row, pl.ds(col, CHUNK)], buf.at[b], sems[b])

    handles = [copy_in(0, 0), copy_in(1, 1)]
    for ci in range(NCHUNK):
        b = ci % 2
        handles[b].wait()

        @plsc.parallel_loop(0, VECS // unroll, unroll=unroll)
        def _(i, _b=b):
            for u in range(unroll):
                compute_vec(u, buf[_b, pl.ds((i * unroll + u) * 16, 16)])

        if ci + 2 < NCHUNK:
            handles[b] = copy_in(ci + 2, b)


def _pass1_body(x_hbm, out_hbm, hist, buf, sem0, sem1):
    wid = lax.axis_index("s") * NC + lax.axis_index("c")
    lane_base = lax.iota(jnp.int32, 16) * B1
    ones = jnp.ones((16,), jnp.int32)
    _zero(hist, H1)

    def compute_vec(u, v):
        key = _mono_key(v)
        bucket = plsc.bitcast(key >> np.uint32(SHIFT1), jnp.int32)
        plsc.addupdate_scatter(hist, [(lane_base + u * 16 * B1) + bucket],
                               ones)

    _stream_chunks(x_hbm, buf, (sem0, sem1), wid, U1, compute_vec)
    _fold_rows(hist, 0, U1 * 16, B1)
    pltpu.sync_copy(hist.at[pl.ds(0, B1)], out_hbm.at[pl.ds(wid * B1, B1)])


def _pass2_body(x_hbm, pref_hbm, out_hbm, hist, buf, lut, pbuf, sem0, sem1):
    wid = lax.axis_index("s") * NC + lax.axis_index("c")
    lane = lax.iota(jnp.int32, 16)
    lane_base = lane * B2
    ones = jnp.ones((16,), jnp.int32)
    _zero(hist, NT * U2 * REG)
    pltpu.sync_copy(pref_hbm, pbuf)

    trash = jnp.full((16,), NT * U2 * REG, jnp.int32)

    @plsc.parallel_loop(0, B1 // 16)
    def _(i):
        lut[pl.ds(i * 16, 16)] = trash

    lane0 = lane == 0
    for t in range(NT - 1, -1, -1):
        plsc.store_scatter(lut, [pbuf[t]],
                           jnp.full((16,), t * U2 * REG, jnp.int32),
                           mask=lane0)

    def compute_vec(u, v):
        key = _mono_key(v)
        hi = plsc.bitcast(key >> np.uint32(SHIFT1), jnp.int32)
        bucket = plsc.bitcast(
            (key >> np.uint32(SHIFT2)) & np.uint32(B2 - 1), jnp.int32)
        base = plsc.load_gather(lut, [hi])
        plsc.addupdate_scatter(
            hist, [base + ((lane_base + u * REG) + bucket)], ones)

    _stream_chunks(x_hbm, buf, (sem0, sem1), wid, U2, compute_vec)
    for t in range(NT):
        _fold_rows(hist, t * U2 * REG, U2 * 16, B2)
        pltpu.sync_copy(hist.at[pl.ds(t * U2 * REG, B2)],
                        out_hbm.at[pl.ds((wid * NT + t) * B2, B2)])


_mesh = plsc.VectorSubcoreMesh(core_axis_name="c", subcore_axis_name="s")
_params = pltpu.CompilerParams(needs_layout_passes=False)

_pass1 = functools.partial(
    pl.kernel,
    out_type=jax.ShapeDtypeStruct((NW * B1,), jnp.int32),
    scratch_types=[
        pltpu.VMEM((H1,), jnp.int32),
        pltpu.VMEM((2, CHUNK), jnp.float32),
        pltpu.SemaphoreType.DMA,
        pltpu.SemaphoreType.DMA,
    ],
    mesh=_mesh,
    compiler_params=_params,
)(_pass1_body)

_pass2 = functools.partial(
    pl.kernel,
    out_type=jax.ShapeDtypeStruct((NW * NT * B2,), jnp.int32),
    scratch_types=[
        pltpu.VMEM((H2,), jnp.int32),
        pltpu.VMEM((2, CHUNK), jnp.float32),
        pltpu.VMEM((B1,), jnp.int32),
        pltpu.VMEM((NT, 16), jnp.int32),
        pltpu.SemaphoreType.DMA,
        pltpu.SemaphoreType.DMA,
    ],
    mesh=_mesh,
    compiler_params=_params,
)(_pass2_body)

_POS_LO = 0.05 * (N - 1)
_POS_HI = 0.95 * (N - 1)
_K_LO = int(_POS_LO)
_K_HI = int(_POS_HI)
_F_LO = _POS_LO - _K_LO
_F_HI = _POS_HI - _K_HI
_RANKS = (_K_LO, _K_LO + 1, _K_HI, _K_HI + 1)


def kernel(x, low, high):
    x = lax.stop_gradient(x)

    out1 = _pass1(x)
    hist1 = out1.reshape(NW, B1).sum(axis=0)
    c1 = jnp.cumsum(hist1)
    ranks = jnp.array(_RANKS, jnp.int32)
    p1 = jnp.sum(c1[None, :] <= ranks[:, None], axis=1).astype(jnp.int32)
    below = jnp.where(p1 > 0, c1[jnp.maximum(p1 - 1, 0)], 0)
    r = ranks - below

    prefs = jnp.broadcast_to(p1[:, None], (NT, 16)).astype(jnp.int32)
    out2 = _pass2(x, prefs)
    hist2 = out2.reshape(NW, NT, B2).sum(axis=0)
    first = jnp.argmax(p1[None, :] == p1[:, None], axis=1)
    hist_eff = hist2[first]
    c2 = jnp.cumsum(hist_eff, axis=1)
    b2 = jnp.sum(c2 <= r[:, None], axis=1).astype(jnp.uint32)

    key_mid = ((((p1.astype(jnp.uint32) << 9) | b2) << 13)
               | jnp.uint32(1 << 12))
    orig = jnp.where(key_mid >= _SIGN, key_mid ^ _SIGN, ~key_mid)
    vals = lax.bitcast_convert_type(orig, jnp.float32)

    q_lo = vals[0] + jnp.float32(_F_LO) * (vals[1] - vals[0])
    q_hi = vals[2] + jnp.float32(_F_HI) * (vals[3] - vals[2])

    decay = jnp.float32(0.99)
    new_low = decay * low + (1.0 - decay) * q_lo
    new_high = decay * high + (1.0 - decay) * q_hi
    inverse_scale = jnp.maximum(jnp.float32(1.0), new_high - new_low)
    return (new_low, inverse_scale)


# speedup vs baseline: 49.8378x; 1.0151x over previous
import functools

import jax
import jax.numpy as jnp
import numpy as np
from jax import lax
from jax.experimental import pallas as pl
from jax.experimental.pallas import tpu as pltpu
from jax.experimental.pallas import tpu_sc as plsc

ROWS, COLS = 128, 32768
N = ROWS * COLS
NC, NS = 2, 16
NW = NC * NS
PER_W = N // NW
CHUNK = 16384
NCHUNK = PER_W // CHUNK
ROWCH = COLS // CHUNK
VECS = CHUNK // 16

B1 = 1024
SHIFT1 = 22
U1 = 4
H1 = U1 * 16 * B1

B2 = 512
SHIFT2 = 13
U2 = 2
NT = 4
REG = 16 * B2
H2 = (NT + 1) * U2 * REG

_SIGN = np.uint32(0x80000000)
_MININT = np.int32(-0x80000000)


def _mono_key(v):
    ki = plsc.bitcast(v, jnp.int32)
    flip = (ki >> 31) | _MININT
    return plsc.bitcast(ki ^ flip, jnp.uint32)


def _zero(ref, nwords):
    z = jnp.zeros((16,), jnp.int32)

    @plsc.parallel_loop(0, nwords // 16)
    def _(i):
        ref[pl.ds(i * 16, 16)] = z


def _fold_rows(hist, base, nrows, width):
    @plsc.parallel_loop(0, width // 16)
    def _(j):
        off = base + j * 16
        acc = hist[pl.ds(off, 16)]
        for r in range(1, nrows):
            acc = acc + hist[pl.ds(off + r * width, 16)]
        hist[pl.ds(off, 16)] = acc


def _stream_chunks(x_hbm, buf, sems, wid, unroll, compute_vec):
    def copy_in(ci, b):
        row = wid * (NCHUNK // ROWCH) + ci // ROWCH
        col = (ci % ROWCH) * CHUNK
        return pltpu.async_copy(
            x_hbm.at[row, pl.ds(col, CHUNK)], buf.at[b], sems[b])

    handles = [copy_in(0, 0), copy_in(1, 1)]
    for ci in range(NCHUNK):
        b = ci % 2
        handles[b].wait()

        @plsc.parallel_loop(0, VECS // unroll, unroll=8 // unroll)
        def _(i, _b=b):
            for u in range(unroll):
                compute_vec(u, buf[_b, pl.ds((i * unroll + u) * 16, 16)])

        if ci + 2 < NCHUNK:
            handles[b] = copy_in(ci + 2, b)


def _pass1_body(x_hbm, out_hbm, hist, buf, sem0, sem1):
    wid = lax.axis_index("s") * NC + lax.axis_index("c")
    lane_base = lax.iota(jnp.int32, 16) * B1
    ones = jnp.ones((16,), jnp.int32)
    _zero(hist, H1)

    def compute_vec(u, v):
        key = _mono_key(v)
        bucket = plsc.bitcast(key >> np.uint32(SHIFT1), jnp.int32)
        plsc.addupdate_scatter(hist, [(lane_base + u * 16 * B1) + bucket],
                               ones)

    _stream_chunks(x_hbm, buf, (sem0, sem1), wid, U1, compute_vec)
    _fold_rows(hist, 0, U1 * 16, B1)
    pltpu.sync_copy(hist.at[pl.ds(0, B1)], out_hbm.at[pl.ds(wid * B1, B1)])


def _pass2_body(x_hbm, pref_hbm, out_hbm, hist, buf, lut, pbuf, sem0, sem1):
    wid = lax.axis_index("s") * NC + lax.axis_index("c")
    lane = lax.iota(jnp.int32, 16)
    lane_base = lane * B2
    ones = jnp.ones((16,), jnp.int32)
    _zero(hist, NT * U2 * REG)
    pltpu.sync_copy(pref_hbm, pbuf)

    trash = jnp.full((16,), NT * U2 * REG, jnp.int32)

    @plsc.parallel_loop(0, B1 // 16)
    def _(i):
        lut[pl.ds(i * 16, 16)] = trash

    lane0 = lane == 0
    for t in range(NT - 1, -1, -1):
        plsc.store_scatter(lut, [pbuf[t]],
                           jnp.full((16,), t * U2 * REG, jnp.int32),
                           mask=lane0)

    def compute_vec(u, v):
        key = _mono_key(v)
        hi = plsc.bitcast(key >> np.uint32(SHIFT1), jnp.int32)
        bucket = plsc.bitcast(
            (key >> np.uint32(SHIFT2)) & np.uint32(B2 - 1), jnp.int32)
        base = plsc.load_gather(lut, [hi])
        plsc.addupdate_scatter(
            hist, [base + ((lane_base + u * REG) + bucket)], ones)

    _stream_chunks(x_hbm, buf, (sem0, sem1), wid, U2, compute_vec)
    for t in range(NT):
        _fold_rows(hist, t * U2 * REG, U2 * 16, B2)
        pltpu.sync_copy(hist.at[pl.ds(t * U2 * REG, B2)],
                        out_hbm.at[pl.ds((wid * NT + t) * B2, B2)])


_mesh = plsc.VectorSubcoreMesh(core_axis_name="c", subcore_axis_name="s")
_params = pltpu.CompilerParams(needs_layout_passes=False)

_pass1 = functools.partial(
    pl.kernel,
    out_type=jax.ShapeDtypeStruct((NW * B1,), jnp.int32),
    scratch_types=[
        pltpu.VMEM((H1,), jnp.int32),
        pltpu.VMEM((2, CHUNK), jnp.float32),
        pltpu.SemaphoreType.DMA,
        pltpu.SemaphoreType.DMA,
    ],
    mesh=_mesh,
    compiler_params=_params,
)(_pass1_body)

_pass2 = functools.partial(
    pl.kernel,
    out_type=jax.ShapeDtypeStruct((NW * NT * B2,), jnp.int32),
    scratch_types=[
        pltpu.VMEM((H2,), jnp.int32),
        pltpu.VMEM((2, CHUNK), jnp.float32),
        pltpu.VMEM((B1,), jnp.int32),
        pltpu.VMEM((NT, 16), jnp.int32),
        pltpu.SemaphoreType.DMA,
        pltpu.SemaphoreType.DMA,
    ],
    mesh=_mesh,
    compiler_params=_params,
)(_pass2_body)

_POS_LO = 0.05 * (N - 1)
_POS_HI = 0.95 * (N - 1)
_K_LO = int(_POS_LO)
_K_HI = int(_POS_HI)
_F_LO = _POS_LO - _K_LO
_F_HI = _POS_HI - _K_HI
_RANKS = (_K_LO, _K_LO + 1, _K_HI, _K_HI + 1)


def kernel(x, low, high):
    x = lax.stop_gradient(x)

    out1 = _pass1(x)
    hist1 = out1.reshape(NW, B1).sum(axis=0)
    c1 = jnp.cumsum(hist1)
    ranks = jnp.array(_RANKS, jnp.int32)
    p1 = jnp.sum(c1[None, :] <= ranks[:, None], axis=1).astype(jnp.int32)
    below = jnp.where(p1 > 0, c1[jnp.maximum(p1 - 1, 0)], 0)
    r = ranks - below

    prefs = jnp.broadcast_to(p1[:, None], (NT, 16)).astype(jnp.int32)
    out2 = _pass2(x, prefs)
    hist2 = out2.reshape(NW, NT, B2).sum(axis=0)
    first = jnp.argmax(p1[None, :] == p1[:, None], axis=1)
    hist_eff = hist2[first]
    c2 = jnp.cumsum(hist_eff, axis=1)
    b2 = jnp.sum(c2 <= r[:, None], axis=1).astype(jnp.uint32)

    key_mid = ((((p1.astype(jnp.uint32) << 9) | b2) << 13)
               | jnp.uint32(1 << 12))
    orig = jnp.where(key_mid >= _SIGN, key_mid ^ _SIGN, ~key_mid)
    vals = lax.bitcast_convert_type(orig, jnp.float32)

    q_lo = vals[0] + jnp.float32(_F_LO) * (vals[1] - vals[0])
    q_hi = vals[2] + jnp.float32(_F_HI) * (vals[3] - vals[2])

    decay = jnp.float32(0.99)
    new_low = decay * low + (1.0 - decay) * q_lo
    new_high = decay * high + (1.0 - decay) * q_hi
    inverse_scale = jnp.maximum(jnp.float32(1.0), new_high - new_low)
    return (new_low, inverse_scale)
